# baseline (speedup 1.0000x reference)
import functools

import numpy as np
import jax
import jax.numpy as jnp
from jax.experimental import pallas as pl
from jax.experimental.pallas import tpu as pltpu

_VGG19_STD = np.asarray([0.229, 0.224, 0.225], dtype=np.float32)
_W_C = (np.float32(0.25) / (_VGG19_STD * _VGG19_STD)).astype(np.float32)


def _wsq_rowsum_kernel(x_ref, y_ref, o_ref, *, bm, sub, w0, w1, w2):
    i = pl.program_id(0)
    d = x_ref[...] - y_ref[...]
    s = jnp.sum(d * d, axis=-1, keepdims=True)
    r = jax.lax.broadcasted_iota(jnp.int32, (bm, 1), 0) + i * bm
    c = (r // sub) % 3
    w = jnp.where(c == 0, w0, jnp.where(c == 1, w1, w2))
    o_ref[...] = s * w


def _weighted_sq_rowsums(x2, y2, sub, w0, w1, w2):
    m2, lanes = x2.shape
    bm = m2
    for cand in (192, 96, 48, 24, 8):
        if m2 % cand == 0 and (m2 // cand) % 2 == 0:
            bm = cand
            break
    grid = m2 // bm

    block_in = bm * lanes * 4
    vmem_limit = int(min(2 * 2 * block_in + 4 * bm * 4 + (2 << 20), 60 << 20))

    body = functools.partial(_wsq_rowsum_kernel, bm=bm, sub=sub,
                             w0=w0, w1=w1, w2=w2)
    out = pl.pallas_call(
        body,
        out_shape=jax.ShapeDtypeStruct((m2, 1), jnp.float32),
        grid=(grid,),
        in_specs=[
            pl.BlockSpec((bm, lanes), lambda i: (i, 0)),
            pl.BlockSpec((bm, lanes), lambda i: (i, 0)),
        ],
        out_specs=pl.BlockSpec((bm, 1), lambda i: (i, 0)),
        compiler_params=pltpu.CompilerParams(
            dimension_semantics=("parallel",),
            vmem_limit_bytes=vmem_limit,
        ),
        cost_estimate=pl.CostEstimate(
            flops=3 * m2 * lanes,
            transcendentals=0,
            bytes_accessed=2 * m2 * lanes * 4 + m2 * 4,
        ),
    )(x2, y2)
    return out


def kernel(x, y):
    n, c_in, h, w = x.shape
    hw = h * w

    if c_in == 3:
        w0, w1, w2 = float(_W_C[0]), float(_W_C[1]), float(_W_C[2])
    else:
        ws = float(np.float32(_W_C[0] + _W_C[1] + _W_C[2]))
        w0 = w1 = w2 = ws

    lanes = hw
    for cand in (8192, 4096, 2048, 1024, 512, 256, 128):
        if hw >= cand and hw % cand == 0:
            lanes = cand
            break
    sub = hw // lanes

    m2 = n * c_in * sub
    x2 = x.reshape(m2, lanes)
    y2 = y.reshape(m2, lanes)

    row_sums = _weighted_sq_rowsums(x2, y2, sub, w0, w1, w2)
    scale = np.float32(1.0) / (np.float32(3.0) * np.float32(hw) * np.float32(n))
    return jnp.sum(row_sums) * scale


# speedup vs baseline: 1.0077x; 1.0077x over previous
import functools

import numpy as np
import jax
import jax.numpy as jnp
from jax.experimental import pallas as pl
from jax.experimental.pallas import tpu as pltpu

_VGG19_STD = np.asarray([0.229, 0.224, 0.225], dtype=np.float32)
_W_C = (np.float32(0.25) / (_VGG19_STD * _VGG19_STD)).astype(np.float32)


def _wsq_rowsum_kernel(x_ref, y_ref, o_ref, *, bm, sub, w0, w1, w2):
    i = pl.program_id(0)
    d = x_ref[...] - y_ref[...]
    s = jnp.sum(d * d, axis=-1, keepdims=True)
    r = jax.lax.broadcasted_iota(jnp.int32, (bm, 1), 0) + i * bm
    c = (r // sub) % 3
    w = jnp.where(c == 0, w0, jnp.where(c == 1, w1, w2))
    o_ref[...] = s * w


def _weighted_sq_rowsums(x2, y2, sub, w0, w1, w2):
    m2, lanes = x2.shape
    bm = m2
    for cand in (96, 192, 48, 24, 8):
        if m2 % cand == 0 and (m2 // cand) % 2 == 0:
            bm = cand
            break
    grid = m2 // bm

    block_in = bm * lanes * 4
    vmem_limit = int(min(2 * 2 * block_in + 4 * bm * 4 + (2 << 20), 60 << 20))

    body = functools.partial(_wsq_rowsum_kernel, bm=bm, sub=sub,
                             w0=w0, w1=w1, w2=w2)
    out = pl.pallas_call(
        body,
        out_shape=jax.ShapeDtypeStruct((m2, 1), jnp.float32),
        grid=(grid,),
        in_specs=[
            pl.BlockSpec((bm, lanes), lambda i: (i, 0)),
            pl.BlockSpec((bm, lanes), lambda i: (i, 0)),
        ],
        out_specs=pl.BlockSpec((bm, 1), lambda i: (i, 0)),
        compiler_params=pltpu.CompilerParams(
            dimension_semantics=("parallel",),
            vmem_limit_bytes=vmem_limit,
        ),
        cost_estimate=pl.CostEstimate(
            flops=3 * m2 * lanes,
            transcendentals=0,
            bytes_accessed=2 * m2 * lanes * 4 + m2 * 4,
        ),
    )(x2, y2)
    return out


def kernel(x, y):
    n, c_in, h, w = x.shape
    hw = h * w

    if c_in == 3:
        w0, w1, w2 = float(_W_C[0]), float(_W_C[1]), float(_W_C[2])
    else:
        ws = float(np.float32(_W_C[0] + _W_C[1] + _W_C[2]))
        w0 = w1 = w2 = ws

    lanes = hw
    for cand in (8192, 4096, 2048, 1024, 512, 256, 128):
        if hw >= cand and hw % cand == 0:
            lanes = cand
            break
    sub = hw // lanes

    m2 = n * c_in * sub
    x2 = x.reshape(m2, lanes)
    y2 = y.reshape(m2, lanes)

    row_sums = _weighted_sq_rowsums(x2, y2, sub, w0, w1, w2)
    scale = np.float32(1.0) / (np.float32(3.0) * np.float32(hw) * np.float32(n))
    return jnp.sum(row_sums) * scale
